# baseline (speedup 1.0000x reference)
import functools

import jax
import jax.numpy as jnp
from jax.experimental import pallas as pl
from jax.experimental.pallas import tpu as pltpu

_ROWS = 16384
_D_IN = 512
_D_HID = 128
_BLOCK_R = 4096


def _fused_ae_kernel(x_ref, we_ref, be_ref, wd_ref, bd_ref, out_ref):
    xb = x_ref[...].astype(jnp.bfloat16)
    z = jnp.dot(xb, we_ref[...], preferred_element_type=jnp.float32)
    z = (z + be_ref[...]).astype(jnp.bfloat16)
    r = jnp.dot(z, wd_ref[...], preferred_element_type=jnp.float32)
    out_ref[...] = r + bd_ref[...]


@functools.partial(jax.jit, static_argnames=())
def kernel(x, We, be, Wd, bd):
    web = We.astype(jnp.bfloat16)
    wdb = Wd.astype(jnp.bfloat16)
    be2 = be.reshape(1, _D_HID)
    bd2 = bd.reshape(1, _D_IN)
    grid = (_ROWS // _BLOCK_R,)
    return pl.pallas_call(
        _fused_ae_kernel,
        grid=grid,
        in_specs=[
            pl.BlockSpec((_BLOCK_R, _D_IN), lambda i: (i, 0)),
            pl.BlockSpec((_D_IN, _D_HID), lambda i: (0, 0)),
            pl.BlockSpec((1, _D_HID), lambda i: (0, 0)),
            pl.BlockSpec((_D_HID, _D_IN), lambda i: (0, 0)),
            pl.BlockSpec((1, _D_IN), lambda i: (0, 0)),
        ],
        out_specs=pl.BlockSpec((_BLOCK_R, _D_IN), lambda i: (i, 0)),
        out_shape=jax.ShapeDtypeStruct((_ROWS, _D_IN), jnp.float32),
        compiler_params=pltpu.CompilerParams(
            dimension_semantics=("parallel",)),
    )(x, web, be2, wdb, bd2)


# speedup vs baseline: 1.1780x; 1.1780x over previous
import functools

import jax
import jax.numpy as jnp
from jax.experimental import pallas as pl
from jax.experimental.pallas import tpu as pltpu

_ROWS = 16384
_D_IN = 512
_D_HID = 128
_BLOCK_R = 4096


def _fused_ae_kernel(x_ref, we_ref, be_ref, wd_ref, bd_ref, out_ref):
    w = jnp.dot(we_ref[...].astype(jnp.bfloat16),
                wd_ref[...].astype(jnp.bfloat16),
                preferred_element_type=jnp.float32).astype(jnp.bfloat16)
    c = jnp.dot(be_ref[...], wd_ref[...],
                preferred_element_type=jnp.float32) + bd_ref[...]
    r = jnp.dot(x_ref[...].astype(jnp.bfloat16), w,
                preferred_element_type=jnp.float32)
    out_ref[...] = r + c


@functools.partial(jax.jit, static_argnames=())
def kernel(x, We, be, Wd, bd):
    be2 = be.reshape(1, _D_HID)
    bd2 = bd.reshape(1, _D_IN)
    grid = (_ROWS // _BLOCK_R,)
    return pl.pallas_call(
        _fused_ae_kernel,
        grid=grid,
        in_specs=[
            pl.BlockSpec((_BLOCK_R, _D_IN), lambda i: (i, 0)),
            pl.BlockSpec((_D_IN, _D_HID), lambda i: (0, 0)),
            pl.BlockSpec((1, _D_HID), lambda i: (0, 0)),
            pl.BlockSpec((_D_HID, _D_IN), lambda i: (0, 0)),
            pl.BlockSpec((1, _D_IN), lambda i: (0, 0)),
        ],
        out_specs=pl.BlockSpec((_BLOCK_R, _D_IN), lambda i: (i, 0)),
        out_shape=jax.ShapeDtypeStruct((_ROWS, _D_IN), jnp.float32),
        compiler_params=pltpu.CompilerParams(
            dimension_semantics=("parallel",)),
    )(x, We, be2, Wd, bd2)
